# Initial kernel scaffold; baseline (speedup 1.0000x reference)
import jax
import jax.numpy as jnp
from jax.experimental import pallas as pl


def kernel(pos_input, pos_item, neg_item, input_emb, item_emb, W1, b1, W2, b2):
    raise NotImplementedError("write your pallas kernel here")


# speedup vs baseline: 3.1876x; 3.1876x over previous
import functools

import jax
import jax.numpy as jnp
from jax.experimental import pallas as pl
from jax.experimental.pallas import tpu as pltpu
from jax.experimental.pallas import tpu_sc as plsc


def _sc_gather(table, indices, window):
    n = indices.shape[0]
    d = table.shape[1]
    indices = indices.reshape(1, n)
    mesh = plsc.VectorSubcoreMesh(core_axis_name="core", subcore_axis_name="subcore")

    @functools.partial(
        pl.kernel,
        out_type=jax.ShapeDtypeStruct((n, d), table.dtype),
        mesh=mesh,
    )
    def gather_kernel(tbl_hbm, idx_hbm, out_hbm):
        def body(idx_vmem, out_vmem):
            pltpu.sync_copy(tbl_hbm.at[idx_vmem.at[0]], out_vmem)

        pltpu.emit_pipeline(
            body,
            grid=(n // window,),
            in_specs=[pl.BlockSpec((1, window), index_map=lambda i: (0, i))],
            out_specs=[pl.BlockSpec((window, d), index_map=lambda i: (i, 0))],
            core_axis_name=("core", "subcore"),
            dimension_semantics=(pltpu.PARALLEL,),
        )(idx_hbm, out_hbm)

    return gather_kernel(table, indices)


def _mlp_body(x_ref, w1_ref, b1_ref, w2_ref, b2_ref, o_ref):
    h = jnp.dot(x_ref[...], w1_ref[...], preferred_element_type=jnp.float32)
    h = jnp.maximum(h + b1_ref[...], 0.0)
    o = jnp.dot(h, w2_ref[...], preferred_element_type=jnp.float32)
    o_ref[...] = jnp.maximum(o + b2_ref[...], 0.0)


def _tc_mlp(x, w1, b1, w2, b2, bm):
    b, d_in = x.shape
    h1 = w1.shape[1]
    d_out = w2.shape[1]
    return pl.pallas_call(
        _mlp_body,
        grid=(b // bm,),
        in_specs=[
            pl.BlockSpec((bm, d_in), lambda i: (i, 0)),
            pl.BlockSpec((d_in, h1), lambda i: (0, 0)),
            pl.BlockSpec((1, h1), lambda i: (0, 0)),
            pl.BlockSpec((h1, d_out), lambda i: (0, 0)),
            pl.BlockSpec((1, d_out), lambda i: (0, 0)),
        ],
        out_specs=pl.BlockSpec((bm, d_out), lambda i: (i, 0)),
        out_shape=jax.ShapeDtypeStruct((b, d_out), jnp.float32),
    )(x, w1, b1.reshape(1, h1), w2, b2.reshape(1, d_out))


def _loss_body(n_neg, u_ref, pos_ref, neg_ref, o_ref):
    i = pl.program_id(0)
    u = u_ref[...]
    bm, d = u.shape
    s = jnp.sum(u * pos_ref[...], axis=1)
    s = jnp.clip(s, -10.0, 10.0)
    loss_pos = jnp.log1p(jnp.exp(-s))
    neg = neg_ref[...].reshape(bm, n_neg, d)
    ns = jnp.sum(neg * u[:, None, :], axis=2)
    ns = jnp.clip(ns, -10.0, 10.0)
    loss_neg = jnp.sum(jnp.log1p(jnp.exp(ns)), axis=1)
    part = jnp.sum(loss_pos + loss_neg)[None, None]

    @pl.when(i == 0)
    def _():
        o_ref[...] = jnp.zeros_like(o_ref)

    o_ref[...] += part


def _tc_loss(u, pos_rows, neg_rows, n_neg, bm):
    b, d = u.shape
    return pl.pallas_call(
        functools.partial(_loss_body, n_neg),
        grid=(b // bm,),
        in_specs=[
            pl.BlockSpec((bm, d), lambda i: (i, 0)),
            pl.BlockSpec((bm, d), lambda i: (i, 0)),
            pl.BlockSpec((bm * n_neg, d), lambda i: (i, 0)),
        ],
        out_specs=pl.BlockSpec((1, 1), lambda i: (0, 0)),
        out_shape=jax.ShapeDtypeStruct((1, 1), jnp.float32),
    )(u, pos_rows, neg_rows)


def kernel(pos_input, pos_item, neg_item, input_emb, item_emb, W1, b1, W2, b2):
    b = pos_input.shape[0]
    n_neg = neg_item.shape[1]

    item_idx = jnp.concatenate(
        [pos_item.astype(jnp.int32), neg_item.reshape(-1).astype(jnp.int32)]
    )
    x = _sc_gather(input_emb, pos_input.astype(jnp.int32), window=128)
    items = _sc_gather(item_emb, item_idx, window=128)

    u = _tc_mlp(x, W1, b1, W2, b2, bm=512)
    total = _tc_loss(u, items[:b], items[b:], n_neg, bm=512)
    return (total[0, 0] / b).astype(jnp.float32)


# speedup vs baseline: 5.7079x; 1.7906x over previous
import functools

import jax
import jax.numpy as jnp
from jax import lax
from jax.experimental import pallas as pl
from jax.experimental.pallas import tpu as pltpu
from jax.experimental.pallas import tpu_sc as plsc

_N_CHUNKS = 2
_WINDOW = 256
_BM_MLP = 512
_BM_LOSS = 512


def _sc_gather(table, indices, window):
    n = indices.shape[0]
    d = table.shape[1]
    indices = indices.reshape(1, n)
    mesh = plsc.VectorSubcoreMesh(core_axis_name="core", subcore_axis_name="subcore")

    @functools.partial(
        pl.kernel,
        out_type=jax.ShapeDtypeStruct((n, d), table.dtype),
        mesh=mesh,
    )
    def gather_kernel(tbl_hbm, idx_hbm, out_hbm):
        def body(idx_vmem, out_vmem):
            pltpu.sync_copy(tbl_hbm.at[idx_vmem.at[0]], out_vmem)

        pltpu.emit_pipeline(
            body,
            grid=(n // window,),
            in_specs=[pl.BlockSpec((1, window), index_map=lambda i: (0, i))],
            out_specs=[pl.BlockSpec((window, d), index_map=lambda i: (i, 0))],
            core_axis_name=("core", "subcore"),
            dimension_semantics=(pltpu.PARALLEL,),
        )(idx_hbm, out_hbm)

    return gather_kernel(table, indices)


def _mlp_body(x_ref, w1_ref, b1_ref, w2_ref, b2_ref, o_ref):
    h = jnp.dot(x_ref[...], w1_ref[...], preferred_element_type=jnp.float32)
    h = jnp.maximum(h + b1_ref[...], 0.0)
    o = jnp.dot(h, w2_ref[...], preferred_element_type=jnp.float32)
    o_ref[...] = jnp.maximum(o + b2_ref[...], 0.0)


def _tc_mlp(x, w1, b1, w2, b2, bm):
    b, d_in = x.shape
    h1 = w1.shape[1]
    d_out = w2.shape[1]
    return pl.pallas_call(
        _mlp_body,
        grid=(b // bm,),
        in_specs=[
            pl.BlockSpec((bm, d_in), lambda i: (i, 0)),
            pl.BlockSpec((d_in, h1), lambda i: (0, 0)),
            pl.BlockSpec((1, h1), lambda i: (0, 0)),
            pl.BlockSpec((h1, d_out), lambda i: (0, 0)),
            pl.BlockSpec((1, d_out), lambda i: (0, 0)),
        ],
        out_specs=pl.BlockSpec((bm, d_out), lambda i: (i, 0)),
        out_shape=jax.ShapeDtypeStruct((b, d_out), jnp.float32),
    )(x, w1, b1.reshape(1, h1), w2, b2.reshape(1, d_out))


def _loss_body(n_neg, u_ref, pos_ref, neg_ref, o_ref):
    i = pl.program_id(0)
    u = u_ref[...]
    s = jnp.sum(u * pos_ref[...], axis=1)
    acc = jnp.log1p(jnp.exp(-jnp.clip(s, -10.0, 10.0)))
    for nn in range(n_neg):
        ns = jnp.sum(neg_ref[nn] * u, axis=1)
        acc = acc + jnp.log1p(jnp.exp(jnp.clip(ns, -10.0, 10.0)))
    part = jnp.sum(acc)[None, None]

    @pl.when(i == 0)
    def _():
        o_ref[...] = jnp.zeros_like(o_ref)

    o_ref[...] += part


def _tc_loss_chunk(u, pos_rows, neg3, chunk_idx, chunk_b, n_neg, bm):
    d = u.shape[1]
    base = chunk_idx * (chunk_b // bm)
    return pl.pallas_call(
        functools.partial(_loss_body, n_neg),
        grid=(chunk_b // bm,),
        in_specs=[
            pl.BlockSpec((bm, d), lambda i: (base + i, 0)),
            pl.BlockSpec((bm, d), lambda i: (base + i, 0)),
            pl.BlockSpec((n_neg, bm, d), lambda i: (0, i, 0)),
        ],
        out_specs=pl.BlockSpec((1, 1), lambda i: (0, 0)),
        out_shape=jax.ShapeDtypeStruct((1, 1), jnp.float32),
    )(u, pos_rows, neg3)


def kernel(pos_input, pos_item, neg_item, input_emb, item_emb, W1, b1, W2, b2):
    b = pos_input.shape[0]
    n_neg = neg_item.shape[1]
    d = item_emb.shape[1]
    nc = _N_CHUNKS
    chunk_b = b // nc

    neg_idx = (
        neg_item.astype(jnp.int32)
        .reshape(nc, chunk_b, n_neg)
        .transpose(0, 2, 1)
        .reshape(nc, n_neg * chunk_b)
    )

    x = _sc_gather(input_emb, pos_input.astype(jnp.int32), window=_WINDOW)
    u = _tc_mlp(x, W1, b1, W2, b2, bm=_BM_MLP)

    pos_idx = lax.optimization_barrier((pos_item.astype(jnp.int32), x))[0]
    pos_rows = _sc_gather(item_emb, pos_idx, window=_WINDOW)

    chain = pos_rows
    parts = []
    for c in range(nc):
        idx_c = lax.optimization_barrier((neg_idx[c], chain))[0]
        g = _sc_gather(item_emb, idx_c, window=_WINDOW)
        chain = g
        neg3 = g.reshape(n_neg, chunk_b, d)
        parts.append(_tc_loss_chunk(u, pos_rows, neg3, c, chunk_b, n_neg, bm=_BM_LOSS))

    total = sum(p[0, 0] for p in parts)
    return (total / b).astype(jnp.float32)
